# baseline (speedup 1.0000x reference)
import jax
import jax.numpy as jnp
import numpy as np
from jax import lax
from jax.experimental import pallas as pl
from jax.experimental.pallas import tpu as pltpu
from jax.experimental.pallas import tpu_sc as plsc

N = 262144
F = 64
B = 256
NC = 2
NS = 16
NW = NC * NS
SAMP = N // NW
CHUNK = 128
NCHUNK = SAMP // CHUNK
NFG = F // 16
GSZ = 16 * B
NHB = 4 * NFG
ROW_UNROLL = 8
BS = 2048


XTS = 65


def _sc_body(x_hbm, g_hbm, h_hbm, out_hbm, x_buf, x_t, g_v, h_v, *rest):
    hbufs = rest[:NHB]
    sems = rest[NHB:]
    c = lax.axis_index("c")
    s = lax.axis_index("s")
    wid = s * NC + c
    base = wid * SAMP

    def start_x(ci, b):
        pltpu.make_async_copy(
            x_hbm.at[:, pl.ds(base + ci * CHUNK, CHUNK)], x_buf.at[b], sems[b]
        ).start()

    def wait_x(b):
        pltpu.make_async_copy(
            x_hbm.at[:, pl.ds(base, CHUNK)], x_buf.at[b], sems[b]
        ).wait()

    start_x(0, 0)
    start_x(1, 1)
    pltpu.sync_copy(g_hbm.at[pl.ds(base, SAMP)], g_v)
    pltpu.sync_copy(h_hbm.at[pl.ds(base, SAMP)], h_v)

    zeros = jnp.zeros((16,), jnp.float32)

    def zero_body(i, carry):
        for hb in hbufs:
            hb[pl.ds(i * 16, 16)] = zeros
        return carry

    lax.fori_loop(0, GSZ // 16, zero_body, 0)

    lane_off = lax.iota(jnp.int32, 16) * B
    lane16 = lax.iota(jnp.int32, 16)
    tsg = [lane16 * XTS + sg * 16 * XTS for sg in range(CHUNK // 16)]

    def transpose_chunk(b):
        def tbody(f, carry):
            fb = jnp.full((16,), f, jnp.int32)
            for sg in range(CHUNK // 16):
                xv = x_buf[b, f, pl.ds(sg * 16, 16)]
                plsc.store_scatter(x_t, [tsg[sg] + fb], xv)
            return carry

        lax.fori_loop(0, F, tbody, 0)

    def compute_chunk(ci, b):
        transpose_chunk(b)

        def rows_body(r8, carry):
            rows = []
            for u in range(ROW_UNROLL):
                r = r8 * ROW_UNROLL + u
                gi = ci * CHUNK + r
                gidx = jnp.full((16,), gi, jnp.int32)
                gs = plsc.load_gather(g_v, [gidx])
                hs = plsc.load_gather(h_v, [gidx])
                idxs = [
                    x_t[pl.ds(r * XTS + fg * 16, 16)] + lane_off
                    for fg in range(NFG)
                ]
                rows.append((gs, hs, idxs))
            for u, (gs, hs, idxs) in enumerate(rows):
                par = (u % 2) * 2 * NFG
                for fg in range(NFG):
                    plsc.addupdate_scatter(hbufs[par + fg], [idxs[fg]], gs)
                    plsc.addupdate_scatter(hbufs[par + NFG + fg], [idxs[fg]], hs)
            return carry

        lax.fori_loop(0, CHUNK // ROW_UNROLL, rows_body, 0)

    def step_body(si, carry):
        for b in range(2):
            ci = si * 2 + b
            wait_x(b)
            compute_chunk(ci, b)

            @pl.when(ci + 2 < NCHUNK)
            def _():
                start_x(ci + 2, b)

        return carry

    lax.fori_loop(0, NCHUNK // 2, step_body, 0)

    for k, hb in enumerate(hbufs):
        pltpu.sync_copy(hb, out_hbm.at[wid, k])


_sc_hist = pl.kernel(
    _sc_body,
    out_type=jax.ShapeDtypeStruct((NW, NHB, GSZ), jnp.float32),
    mesh=plsc.VectorSubcoreMesh(
        core_axis_name="c", subcore_axis_name="s", num_cores=NC, num_subcores=NS
    ),
    compiler_params=pltpu.CompilerParams(needs_layout_passes=False),
    scratch_types=[
        pltpu.VMEM((2, F, CHUNK), jnp.int32),
        pltpu.VMEM((CHUNK * XTS,), jnp.int32),
        pltpu.VMEM((SAMP,), jnp.float32),
        pltpu.VMEM((SAMP,), jnp.float32),
    ]
    + [pltpu.VMEM((GSZ,), jnp.float32) for _ in range(NHB)]
    + [
        pltpu.SemaphoreType.DMA,
        pltpu.SemaphoreType.DMA,
    ],
)


def _tc_body(p_ref, gl_ref, hl_ref):
    acc = jnp.sum(p_ref[...], axis=0)
    bi = lax.broadcasted_iota(jnp.int32, (B, B), 0)
    ki = lax.broadcasted_iota(jnp.int32, (B, B), 1)
    m = (bi > ki).astype(jnp.float32)
    gl_ref[...] = lax.dot(acc[:F], m, precision=lax.Precision.HIGHEST)
    hl_ref[...] = lax.dot(acc[F:], m, precision=lax.Precision.HIGHEST)


_tc_finish = pl.pallas_call(
    _tc_body,
    out_shape=(
        jax.ShapeDtypeStruct((F, B), jnp.float32),
        jax.ShapeDtypeStruct((F, B), jnp.float32),
    ),
)


@jax.jit
def kernel(X, gradient, hessian):
    partials = _sc_hist(X.T, gradient, hessian)
    gl, hl = _tc_finish(partials.reshape(2 * NW, 2 * F, B))
    return (gl[None], hl[None])


# speedup vs baseline: 1.3857x; 1.3857x over previous
import jax
import jax.numpy as jnp
import numpy as np
from jax import lax
from jax.experimental import pallas as pl
from jax.experimental.pallas import tpu as pltpu
from jax.experimental.pallas import tpu_sc as plsc

N = 262144
F = 64
B = 256
NC = 2
NS = 16
NW = NC * NS
SAMP = N // NW
CHUNK = 128
NCHUNK = SAMP // CHUNK
NFG = F // 16
GSZ = 16 * B
NHB = 4 * NFG
ROW_UNROLL = 8
BS = 2048


XTS = 65


def _sc_body(x_hbm, g_hbm, h_hbm, out_hbm, x_buf, x_t, g_v, h_v, *rest):
    hbufs = rest[:NHB]
    sems = rest[NHB:]
    c = lax.axis_index("c")
    s = lax.axis_index("s")
    wid = s * NC + c
    base = wid * SAMP

    def start_x(ci, b):
        pltpu.make_async_copy(
            x_hbm.at[:, pl.ds(base + ci * CHUNK, CHUNK)], x_buf.at[b], sems[b]
        ).start()

    def wait_x(b):
        pltpu.make_async_copy(
            x_hbm.at[:, pl.ds(base, CHUNK)], x_buf.at[b], sems[b]
        ).wait()

    start_x(0, 0)
    start_x(1, 1)
    pltpu.sync_copy(g_hbm.at[pl.ds(base, SAMP)], g_v)
    pltpu.sync_copy(h_hbm.at[pl.ds(base, SAMP)], h_v)

    zeros = jnp.zeros((16,), jnp.float32)

    def zero_body(i, carry):
        for hb in hbufs:
            hb[pl.ds(i * 16, 16)] = zeros
        return carry

    lax.fori_loop(0, GSZ // 16, zero_body, 0)

    lane_off = lax.iota(jnp.int32, 16) * B
    lane16 = lax.iota(jnp.int32, 16)
    tsg = [lane16 * XTS + sg * 16 * XTS for sg in range(CHUNK // 16)]

    def transpose_chunk(b):
        def tbody(f, carry):
            fb = jnp.full((16,), f, jnp.int32)
            xvs = [x_buf[b, f, pl.ds(sg * 16, 16)] for sg in range(CHUNK // 16)]
            idxv = [tsg[sg] + fb for sg in range(CHUNK // 16)]
            for sg in range(CHUNK // 16):
                plsc.store_scatter(x_t, [idxv[sg]], xvs[sg])
            return carry

        lax.fori_loop(0, F, tbody, 0)

    def compute_chunk(ci, b):
        transpose_chunk(b)

        def rows_body(r8, carry):
            rows = []
            for u in range(ROW_UNROLL):
                r = r8 * ROW_UNROLL + u
                gi = ci * CHUNK + r
                gidx = jnp.full((16,), gi, jnp.int32)
                gs = plsc.load_gather(g_v, [gidx])
                hs = plsc.load_gather(h_v, [gidx])
                idxs = [
                    x_t[pl.ds(r * XTS + fg * 16, 16)] + lane_off
                    for fg in range(NFG)
                ]
                rows.append((gs, hs, idxs))
            for u, (gs, hs, idxs) in enumerate(rows):
                par = (u % 2) * 2 * NFG
                for fg in range(NFG):
                    plsc.addupdate_scatter(hbufs[par + fg], [idxs[fg]], gs)
                    plsc.addupdate_scatter(hbufs[par + NFG + fg], [idxs[fg]], hs)
            return carry

        lax.fori_loop(0, CHUNK // ROW_UNROLL, rows_body, 0)

    def step_body(si, carry):
        for b in range(2):
            ci = si * 2 + b
            wait_x(b)
            compute_chunk(ci, b)

            @pl.when(ci + 2 < NCHUNK)
            def _():
                start_x(ci + 2, b)

        return carry

    lax.fori_loop(0, NCHUNK // 2, step_body, 0)

    for k, hb in enumerate(hbufs):
        pltpu.sync_copy(hb, out_hbm.at[wid, k])


_sc_hist = pl.kernel(
    _sc_body,
    out_type=jax.ShapeDtypeStruct((NW, NHB, GSZ), jnp.float32),
    mesh=plsc.VectorSubcoreMesh(
        core_axis_name="c", subcore_axis_name="s", num_cores=NC, num_subcores=NS
    ),
    compiler_params=pltpu.CompilerParams(needs_layout_passes=False),
    scratch_types=[
        pltpu.VMEM((2, F, CHUNK), jnp.int32),
        pltpu.VMEM((CHUNK * XTS,), jnp.int32),
        pltpu.VMEM((SAMP,), jnp.float32),
        pltpu.VMEM((SAMP,), jnp.float32),
    ]
    + [pltpu.VMEM((GSZ,), jnp.float32) for _ in range(NHB)]
    + [
        pltpu.SemaphoreType.DMA,
        pltpu.SemaphoreType.DMA,
    ],
)


def _tc_body(p_ref, gl_ref, hl_ref):
    acc = jnp.sum(p_ref[...], axis=0)
    bi = lax.broadcasted_iota(jnp.int32, (B, B), 0)
    ki = lax.broadcasted_iota(jnp.int32, (B, B), 1)
    m = (bi > ki).astype(jnp.float32)
    gl_ref[...] = lax.dot(acc[:F], m, precision=lax.Precision.HIGHEST)
    hl_ref[...] = lax.dot(acc[F:], m, precision=lax.Precision.HIGHEST)


_tc_finish = pl.pallas_call(
    _tc_body,
    out_shape=(
        jax.ShapeDtypeStruct((F, B), jnp.float32),
        jax.ShapeDtypeStruct((F, B), jnp.float32),
    ),
)


@jax.jit
def kernel(X, gradient, hessian):
    partials = _sc_hist(X.T, gradient, hessian)
    gl, hl = _tc_finish(partials.reshape(2 * NW, 2 * F, B))
    return (gl[None], hl[None])


# speedup vs baseline: 1.4302x; 1.0321x over previous
import jax
import jax.numpy as jnp
import numpy as np
from jax import lax
from jax.experimental import pallas as pl
from jax.experimental.pallas import tpu as pltpu
from jax.experimental.pallas import tpu_sc as plsc

N = 262144
F = 64
B = 256
NC = 2
NS = 16
NW = NC * NS
SAMP = N // NW
CHUNK = 128
NCHUNK = SAMP // CHUNK
NFG = F // 16
GSZ = 16 * B
NHB = 4 * NFG
ROW_UNROLL = 8
BS = 2048


XTS = 65


def _sc_body(x_hbm, g_hbm, h_hbm, out_hbm, x_buf, x_t, g_v, h_v, *rest):
    hbufs = rest[:NHB]
    sems = rest[NHB:]
    c = lax.axis_index("c")
    s = lax.axis_index("s")
    wid = s * NC + c
    base = wid * SAMP

    def start_x(ci, b):
        pltpu.make_async_copy(
            x_hbm.at[:, pl.ds(base + ci * CHUNK, CHUNK)], x_buf.at[b], sems[b]
        ).start()

    def wait_x(b):
        pltpu.make_async_copy(
            x_hbm.at[:, pl.ds(base, CHUNK)], x_buf.at[b], sems[b]
        ).wait()

    start_x(0, 0)
    start_x(1, 1)
    pltpu.sync_copy(g_hbm.at[pl.ds(base, SAMP)], g_v)
    pltpu.sync_copy(h_hbm.at[pl.ds(base, SAMP)], h_v)

    zeros = jnp.zeros((16,), jnp.float32)

    def zero_body(i, carry):
        for hb in hbufs:
            hb[pl.ds(i * 16, 16)] = zeros
        return carry

    lax.fori_loop(0, GSZ // 16, zero_body, 0)

    lane_off = lax.iota(jnp.int32, 16) * B
    lane16 = lax.iota(jnp.int32, 16)
    tsg = [lane16 * XTS + sg * 16 * XTS for sg in range(CHUNK // 16)]

    def transpose_chunk(b):
        def tbody(f, carry):
            fb = jnp.full((16,), f, jnp.int32)
            xvs = [x_buf[b, f, pl.ds(sg * 16, 16)] for sg in range(CHUNK // 16)]
            idxv = [tsg[sg] + fb for sg in range(CHUNK // 16)]
            for sg in range(CHUNK // 16):
                plsc.store_scatter(x_t, [idxv[sg]], xvs[sg])
            return carry

        lax.fori_loop(0, F, tbody, 0)

    def compute_chunk(ci, b):
        transpose_chunk(b)

        def rows_body(r8, carry):
            rows = []
            for u in range(ROW_UNROLL):
                r = r8 * ROW_UNROLL + u
                gi = ci * CHUNK + r
                gidx = jnp.full((16,), gi, jnp.int32)
                gs = plsc.load_gather(g_v, [gidx])
                hs = plsc.load_gather(h_v, [gidx])
                idxs = [
                    x_t[pl.ds(r * XTS + fg * 16, 16)] + lane_off
                    for fg in range(NFG)
                ]
                rows.append((gs, hs, idxs))
            for u, (gs, hs, idxs) in enumerate(rows):
                par = (u % 2) * 2 * NFG
                for fg in range(NFG):
                    plsc.addupdate_scatter(hbufs[par + fg], [idxs[fg]], gs)
                    plsc.addupdate_scatter(hbufs[par + NFG + fg], [idxs[fg]], hs)
            return carry

        lax.fori_loop(0, CHUNK // ROW_UNROLL, rows_body, 0)

    def step_body(si, carry):
        for b in range(2):
            ci = si * 2 + b
            wait_x(b)
            compute_chunk(ci, b)

            @pl.when(ci + 2 < NCHUNK)
            def _():
                start_x(ci + 2, b)

        return carry

    lax.fori_loop(0, NCHUNK // 2, step_body, 0)

    for k, hb in enumerate(hbufs):
        pltpu.sync_copy(hb, out_hbm.at[wid, k])


_sc_hist = pl.kernel(
    _sc_body,
    out_type=jax.ShapeDtypeStruct((NW, NHB, GSZ), jnp.float32),
    mesh=plsc.VectorSubcoreMesh(
        core_axis_name="c", subcore_axis_name="s", num_cores=NC, num_subcores=NS
    ),
    compiler_params=pltpu.CompilerParams(needs_layout_passes=False),
    scratch_types=[
        pltpu.VMEM((2, F, CHUNK), jnp.int32),
        pltpu.VMEM((CHUNK * XTS,), jnp.int32),
        pltpu.VMEM((SAMP,), jnp.float32),
        pltpu.VMEM((SAMP,), jnp.float32),
    ]
    + [pltpu.VMEM((GSZ,), jnp.float32) for _ in range(NHB)]
    + [
        pltpu.SemaphoreType.DMA,
        pltpu.SemaphoreType.DMA,
    ],
)


def _tc_body(p_ref, o_ref):
    acc = jnp.sum(p_ref[...], axis=0)
    bi = lax.broadcasted_iota(jnp.int32, (B, B), 0)
    ki = lax.broadcasted_iota(jnp.int32, (B, B), 1)
    m = (bi > ki).astype(jnp.float32)
    for L in range(16):
        blk = acc[:, L * B:(L + 1) * B]
        o_ref[pl.ds(L * 8, 8), :] = lax.dot(
            blk, m, precision=lax.Precision.HIGHEST
        )


_tc_finish = pl.pallas_call(
    _tc_body,
    out_shape=jax.ShapeDtypeStruct((2 * F, B), jnp.float32),
)

_PG = np.array([(f % 16) * 8 + f // 16 for f in range(F)], np.int32)
_PH = _PG + 4


@jax.jit
def kernel(X, gradient, hessian):
    partials = _sc_hist(X.T, gradient, hessian)
    out2 = _tc_finish(partials.reshape(2 * NW, NHB // 2, GSZ))
    return (out2[_PG][None], out2[_PH][None])


# speedup vs baseline: 1.4333x; 1.0022x over previous
import jax
import jax.numpy as jnp
import numpy as np
from jax import lax
from jax.experimental import pallas as pl
from jax.experimental.pallas import tpu as pltpu
from jax.experimental.pallas import tpu_sc as plsc

N = 262144
F = 64
B = 256
NC = 2
NS = 16
NW = NC * NS
SAMP = N // NW
CHUNK = 128
NCHUNK = SAMP // CHUNK
NFG = F // 16
GSZ = 16 * B
NHB = 4 * NFG
ROW_UNROLL = 8
BS = 2048


XTS = 65


def _sc_body(x_hbm, g_hbm, h_hbm, out_hbm, x_buf, x_t, g_v, h_v, *rest):
    hbufs = rest[:NHB]
    sems = rest[NHB:]
    c = lax.axis_index("c")
    s = lax.axis_index("s")
    wid = s * NC + c
    base = wid * SAMP

    def start_x(ci, b):
        pltpu.make_async_copy(
            x_hbm.at[:, pl.ds(base + ci * CHUNK, CHUNK)], x_buf.at[b], sems[b]
        ).start()

    def wait_x(b):
        pltpu.make_async_copy(
            x_hbm.at[:, pl.ds(base, CHUNK)], x_buf.at[b], sems[b]
        ).wait()

    start_x(0, 0)
    start_x(1, 1)
    pltpu.sync_copy(g_hbm.at[pl.ds(base, SAMP)], g_v)
    pltpu.sync_copy(h_hbm.at[pl.ds(base, SAMP)], h_v)

    zeros = jnp.zeros((16,), jnp.float32)

    def zero_body(i, carry):
        for hb in hbufs:
            hb[pl.ds(i * 16, 16)] = zeros
        return carry

    lax.fori_loop(0, GSZ // 16, zero_body, 0)

    lane_off = lax.iota(jnp.int32, 16) * B
    lane16 = lax.iota(jnp.int32, 16)
    tsg = [lane16 * XTS + sg * 16 * XTS for sg in range(CHUNK // 16)]

    def transpose_chunk(b):
        def tbody(f2, carry):
            for u in range(2):
                f = f2 * 2 + u
                fb = jnp.full((16,), f, jnp.int32)
                xvs = [
                    x_buf[b, f, pl.ds(sg * 16, 16)]
                    for sg in range(CHUNK // 16)
                ]
                idxv = [tsg[sg] + fb for sg in range(CHUNK // 16)]
                for sg in range(CHUNK // 16):
                    plsc.store_scatter(x_t, [idxv[sg]], xvs[sg])
            return carry

        lax.fori_loop(0, F // 2, tbody, 0)

    def compute_chunk(ci, b):
        transpose_chunk(b)

        def rows_body(r8, carry):
            rows = []
            for u in range(ROW_UNROLL):
                r = r8 * ROW_UNROLL + u
                gi = ci * CHUNK + r
                gidx = jnp.full((16,), gi, jnp.int32)
                gs = plsc.load_gather(g_v, [gidx])
                hs = plsc.load_gather(h_v, [gidx])
                idxs = [
                    x_t[pl.ds(r * XTS + fg * 16, 16)] + lane_off
                    for fg in range(NFG)
                ]
                rows.append((gs, hs, idxs))
            for u, (gs, hs, idxs) in enumerate(rows):
                par = (u % 2) * 2 * NFG
                for fg in range(NFG):
                    plsc.addupdate_scatter(hbufs[par + fg], [idxs[fg]], gs)
                    plsc.addupdate_scatter(hbufs[par + NFG + fg], [idxs[fg]], hs)
            return carry

        lax.fori_loop(0, CHUNK // ROW_UNROLL, rows_body, 0)

    def step_body(si, carry):
        for b in range(2):
            ci = si * 2 + b
            wait_x(b)
            compute_chunk(ci, b)

            @pl.when(ci + 2 < NCHUNK)
            def _():
                start_x(ci + 2, b)

        return carry

    lax.fori_loop(0, NCHUNK // 2, step_body, 0)

    for k, hb in enumerate(hbufs):
        pltpu.sync_copy(hb, out_hbm.at[wid, k])


_sc_hist = pl.kernel(
    _sc_body,
    out_type=jax.ShapeDtypeStruct((NW, NHB, GSZ), jnp.float32),
    mesh=plsc.VectorSubcoreMesh(
        core_axis_name="c", subcore_axis_name="s", num_cores=NC, num_subcores=NS
    ),
    compiler_params=pltpu.CompilerParams(needs_layout_passes=False),
    scratch_types=[
        pltpu.VMEM((2, F, CHUNK), jnp.int32),
        pltpu.VMEM((CHUNK * XTS,), jnp.int32),
        pltpu.VMEM((SAMP,), jnp.float32),
        pltpu.VMEM((SAMP,), jnp.float32),
    ]
    + [pltpu.VMEM((GSZ,), jnp.float32) for _ in range(NHB)]
    + [
        pltpu.SemaphoreType.DMA,
        pltpu.SemaphoreType.DMA,
    ],
)


def _tc_body(p_ref, o_ref):
    acc = jnp.sum(p_ref[...], axis=0)
    bi = lax.broadcasted_iota(jnp.int32, (B, B), 0)
    ki = lax.broadcasted_iota(jnp.int32, (B, B), 1)
    m = (bi > ki).astype(jnp.float32)
    for L in range(16):
        blk = acc[:, L * B:(L + 1) * B]
        o_ref[pl.ds(L * 8, 8), :] = lax.dot(
            blk, m, precision=lax.Precision.HIGHEST
        )


_tc_finish = pl.pallas_call(
    _tc_body,
    out_shape=jax.ShapeDtypeStruct((2 * F, B), jnp.float32),
)

_PG = np.array([(f % 16) * 8 + f // 16 for f in range(F)], np.int32)
_PH = _PG + 4


@jax.jit
def kernel(X, gradient, hessian):
    partials = _sc_hist(X.T, gradient, hessian)
    out2 = _tc_finish(partials.reshape(2 * NW, NHB // 2, GSZ))
    return (out2[_PG][None], out2[_PH][None])
